# baseline (device time: 54510 ns/iter reference)
import functools

import jax
import jax.numpy as jnp
from jax import lax
from jax.experimental import pallas as pl
from jax.experimental.pallas import tpu as pltpu

N_DEV = 16
LOG2 = 4
N_LAYERS = 3
N_SLOTS = N_LAYERS * LOG2


def _partner_idx(my_idx, step):
    z = my_idx // 4
    p = my_idx % 4
    g = p ^ (p >> 1)
    x = g & 1
    y = g >> 1
    if step == 0:
        x = 1 - x
    elif step == 1:
        y = 1 - y
    elif step == 2:
        z = z ^ 1
    else:
        z = z ^ 2
    g2 = y * 2 + x
    return z * 4 + (g2 ^ (g2 >> 1))


def kernel(x, Win0, Wout0, Win1, Wout1, Win2, Wout2):
    b, d_shard = x.shape
    h_dim = Win0.shape[1]

    def body(x_ref, wi0, wo0, wi1, wo1, wi2, wo2, out_ref,
             send_buf, recv_buf, send_sems, recv_sems):
        my = lax.axis_index("i")
        partners = [_partner_idx(my, s) for s in range(LOG2)]

        barrier = pltpu.get_barrier_semaphore()
        for s in range(LOG2):
            pl.semaphore_signal(
                barrier, inc=1,
                device_id=(partners[s],),
                device_id_type=pl.DeviceIdType.MESH,
            )
        pl.semaphore_wait(barrier, LOG2)

        x_val = x_ref[...]
        for l, (wi, wo) in enumerate([(wi0, wo0), (wi1, wo1), (wi2, wo2)]):
            acc = jnp.dot(x_val, wi[...], preferred_element_type=jnp.float32)
            for s in range(LOG2):
                slot = l * LOG2 + s
                send_buf[slot, :, :] = acc
                rdma = pltpu.make_async_remote_copy(
                    src_ref=send_buf.at[slot],
                    dst_ref=recv_buf.at[slot],
                    send_sem=send_sems.at[slot],
                    recv_sem=recv_sems.at[slot],
                    device_id=(partners[s],),
                    device_id_type=pl.DeviceIdType.MESH,
                )
                rdma.start()
                rdma.wait()
                acc = acc + recv_buf[slot, :, :]
            x_val = jnp.dot(
                jnp.maximum(acc, 0.0), wo[...],
                preferred_element_type=jnp.float32,
            )
        out_ref[...] = x_val

        @functools.partial(pl.run_scoped, sem=pltpu.SemaphoreType.REGULAR)
        def _(sem):
            for s in range(LOG2):
                pl.semaphore_signal(
                    sem, inc=1,
                    device_id=(partners[s],),
                    device_id_type=pl.DeviceIdType.MESH,
                )
            pl.semaphore_wait(sem, LOG2)

    return pl.pallas_call(
        body,
        out_shape=jax.ShapeDtypeStruct((b, d_shard), jnp.float32),
        in_specs=[pl.BlockSpec(memory_space=pltpu.VMEM)] * 7,
        out_specs=pl.BlockSpec(memory_space=pltpu.VMEM),
        scratch_shapes=[
            pltpu.VMEM((N_SLOTS, b, h_dim), jnp.float32),
            pltpu.VMEM((N_SLOTS, b, h_dim), jnp.float32),
            pltpu.SemaphoreType.DMA((N_SLOTS,)),
            pltpu.SemaphoreType.DMA((N_SLOTS,)),
        ],
        compiler_params=pltpu.CompilerParams(collective_id=0),
    )(x, Win0, Wout0, Win1, Wout1, Win2, Wout2)


# device time: 46743 ns/iter; 1.1662x vs baseline; 1.1662x over previous
import functools

import jax
import jax.numpy as jnp
from jax import lax
from jax.experimental import pallas as pl
from jax.experimental.pallas import tpu as pltpu

N_DEV = 16
N_DIMS = 4
N_LAYERS = 3
N_Q = 4
N_SLOTS = N_LAYERS * N_DIMS * N_Q


def _partner_idx(my_idx, dim):
    z = my_idx // 4
    p = my_idx % 4
    g = p ^ (p >> 1)
    x = g & 1
    y = g >> 1
    if dim == 0:
        x = 1 - x
    elif dim == 1:
        y = 1 - y
    elif dim == 2:
        z = z ^ 1
    else:
        z = z ^ 2
    g2 = y * 2 + x
    return z * 4 + (g2 ^ (g2 >> 1))


def kernel(x, Win0, Wout0, Win1, Wout1, Win2, Wout2):
    b, d_shard = x.shape
    h_dim = Win0.shape[1]
    bq = b // N_Q

    def body(x_ref, wi0, wo0, wi1, wo1, wi2, wo2, out_ref,
             send_buf, recv_buf, send_sems, recv_sems):
        my = lax.axis_index("i")
        partners = [_partner_idx(my, d) for d in range(N_DIMS)]

        barrier = pltpu.get_barrier_semaphore()
        for d in range(N_DIMS):
            pl.semaphore_signal(
                barrier, inc=1,
                device_id=(partners[d],),
                device_id_type=pl.DeviceIdType.MESH,
            )
        pl.semaphore_wait(barrier, N_DIMS)

        x_val = x_ref[...]
        for l, (wi, wo) in enumerate([(wi0, wo0), (wi1, wo1), (wi2, wo2)]):
            partial = jnp.dot(x_val, wi[...], preferred_element_type=jnp.float32)
            acc = [partial[q * bq:(q + 1) * bq, :] for q in range(N_Q)]
            for s in range(N_DIMS):
                rdmas = []
                for q in range(N_Q):
                    slot = (l * N_DIMS + s) * N_Q + q
                    dim = (q + s) % N_DIMS
                    send_buf[slot, :, :] = acc[q]
                    rdma = pltpu.make_async_remote_copy(
                        src_ref=send_buf.at[slot],
                        dst_ref=recv_buf.at[slot],
                        send_sem=send_sems.at[slot],
                        recv_sem=recv_sems.at[slot],
                        device_id=(partners[dim],),
                        device_id_type=pl.DeviceIdType.MESH,
                    )
                    rdma.start()
                    rdmas.append(rdma)
                for q in range(N_Q):
                    slot = (l * N_DIMS + s) * N_Q + q
                    rdmas[q].wait()
                    acc[q] = acc[q] + recv_buf[slot, :, :]
            h = jnp.maximum(jnp.concatenate(acc, axis=0), 0.0)
            x_val = jnp.dot(h, wo[...], preferred_element_type=jnp.float32)
        out_ref[...] = x_val

        @functools.partial(pl.run_scoped, sem=pltpu.SemaphoreType.REGULAR)
        def _(sem):
            for d in range(N_DIMS):
                pl.semaphore_signal(
                    sem, inc=1,
                    device_id=(partners[d],),
                    device_id_type=pl.DeviceIdType.MESH,
                )
            pl.semaphore_wait(sem, N_DIMS)

    return pl.pallas_call(
        body,
        out_shape=jax.ShapeDtypeStruct((b, d_shard), jnp.float32),
        in_specs=[pl.BlockSpec(memory_space=pltpu.VMEM)] * 7,
        out_specs=pl.BlockSpec(memory_space=pltpu.VMEM),
        scratch_shapes=[
            pltpu.VMEM((N_SLOTS, bq, h_dim), jnp.float32),
            pltpu.VMEM((N_SLOTS, bq, h_dim), jnp.float32),
            pltpu.SemaphoreType.DMA((N_SLOTS,)),
            pltpu.SemaphoreType.DMA((N_SLOTS,)),
        ],
        compiler_params=pltpu.CompilerParams(collective_id=0),
    )(x, Win0, Wout0, Win1, Wout1, Win2, Wout2)


# device time: 41859 ns/iter; 1.3022x vs baseline; 1.1167x over previous
import functools

import jax
import jax.numpy as jnp
from jax import lax
from jax.experimental import pallas as pl
from jax.experimental.pallas import tpu as pltpu

N_DEV = 16
N_DIMS = 4
N_LAYERS = 3
N_Q = 4
N_SLOTS = N_LAYERS * N_DIMS * N_Q


def _partner_idx(my_idx, dim):
    z = my_idx // 4
    p = my_idx % 4
    g = p ^ (p >> 1)
    x = g & 1
    y = g >> 1
    if dim == 0:
        x = 1 - x
    elif dim == 1:
        y = 1 - y
    elif dim == 2:
        z = z ^ 1
    else:
        z = z ^ 2
    g2 = y * 2 + x
    return z * 4 + (g2 ^ (g2 >> 1))


def kernel(x, Win0, Wout0, Win1, Wout1, Win2, Wout2):
    b, d_shard = x.shape
    h_dim = Win0.shape[1]
    bq = b // N_Q

    def body(x_ref, wi0, wo0, wi1, wo1, wi2, wo2, out_ref,
             send_buf, recv_buf, send_sems, recv_sems):
        my = lax.axis_index("i")
        partners = [_partner_idx(my, d) for d in range(N_DIMS)]

        barrier = pltpu.get_barrier_semaphore()
        for d in range(N_DIMS):
            pl.semaphore_signal(
                barrier, inc=1,
                device_id=(partners[d],),
                device_id_type=pl.DeviceIdType.MESH,
            )
        pl.semaphore_wait(barrier, N_DIMS)

        def make_rdma(l, s, q):
            slot = (l * N_DIMS + s) * N_Q + q
            dim = (q + s) % N_DIMS
            return slot, pltpu.make_async_remote_copy(
                src_ref=send_buf.at[slot],
                dst_ref=recv_buf.at[slot],
                send_sem=send_sems.at[slot],
                recv_sem=recv_sems.at[slot],
                device_id=(partners[dim],),
                device_id_type=pl.DeviceIdType.MESH,
            )

        x_val = x_ref[...]
        for l, (wi, wo) in enumerate([(wi0, wo0), (wi1, wo1), (wi2, wo2)]):
            partial = jnp.dot(x_val, wi[...], preferred_element_type=jnp.float32)
            acc = [partial[q * bq:(q + 1) * bq, :] for q in range(N_Q)]
            for q in range(N_Q):
                slot, rdma = make_rdma(l, 0, q)
                send_buf[slot, :, :] = acc[q]
                rdma.start()
            for s in range(N_DIMS):
                for q in range(N_Q):
                    slot, rdma = make_rdma(l, s, q)
                    rdma.wait()
                    acc[q] = acc[q] + recv_buf[slot, :, :]
                    if s + 1 < N_DIMS:
                        nslot, nrdma = make_rdma(l, s + 1, q)
                        send_buf[nslot, :, :] = acc[q]
                        nrdma.start()
            h = jnp.maximum(jnp.concatenate(acc, axis=0), 0.0)
            x_val = jnp.dot(h, wo[...], preferred_element_type=jnp.float32)
        out_ref[...] = x_val

        @functools.partial(pl.run_scoped, sem=pltpu.SemaphoreType.REGULAR)
        def _(sem):
            for d in range(N_DIMS):
                pl.semaphore_signal(
                    sem, inc=1,
                    device_id=(partners[d],),
                    device_id_type=pl.DeviceIdType.MESH,
                )
            pl.semaphore_wait(sem, N_DIMS)

    return pl.pallas_call(
        body,
        out_shape=jax.ShapeDtypeStruct((b, d_shard), jnp.float32),
        in_specs=[pl.BlockSpec(memory_space=pltpu.VMEM)] * 7,
        out_specs=pl.BlockSpec(memory_space=pltpu.VMEM),
        scratch_shapes=[
            pltpu.VMEM((N_SLOTS, bq, h_dim), jnp.float32),
            pltpu.VMEM((N_SLOTS, bq, h_dim), jnp.float32),
            pltpu.SemaphoreType.DMA((N_SLOTS,)),
            pltpu.SemaphoreType.DMA((N_SLOTS,)),
        ],
        compiler_params=pltpu.CompilerParams(collective_id=0),
    )(x, Win0, Wout0, Win1, Wout1, Win2, Wout2)


# device time: 39897 ns/iter; 1.3663x vs baseline; 1.0492x over previous
import functools

import jax
import jax.numpy as jnp
from jax import lax
from jax.experimental import pallas as pl
from jax.experimental.pallas import tpu as pltpu

N_DEV = 16
N_DIMS = 4
N_LAYERS = 3
N_Q = 4
N_SLOTS = N_LAYERS * N_DIMS * N_Q


def _partner_idx(my_idx, dim):
    z = my_idx // 4
    p = my_idx % 4
    g = p ^ (p >> 1)
    x = g & 1
    y = g >> 1
    if dim == 0:
        x = 1 - x
    elif dim == 1:
        y = 1 - y
    elif dim == 2:
        z = z ^ 1
    else:
        z = z ^ 2
    g2 = y * 2 + x
    return z * 4 + (g2 ^ (g2 >> 1))


def kernel(x, Win0, Wout0, Win1, Wout1, Win2, Wout2):
    b, d_shard = x.shape
    h_dim = Win0.shape[1]
    bq = b // N_Q

    def body(x_ref, wi0, wo0, wi1, wo1, wi2, wo2, out_ref,
             send_buf, recv_buf, send_sems, recv_sems):
        my = lax.axis_index("i")
        partners = [_partner_idx(my, d) for d in range(N_DIMS)]

        barrier = pltpu.get_barrier_semaphore()
        for d in range(N_DIMS):
            pl.semaphore_signal(
                barrier, inc=1,
                device_id=(partners[d],),
                device_id_type=pl.DeviceIdType.MESH,
            )
        pl.semaphore_wait(barrier, N_DIMS)

        def make_rdma(l, s, q):
            slot = (l * N_DIMS + s) * N_Q + q
            dim = (q + s) % N_DIMS
            return slot, pltpu.make_async_remote_copy(
                src_ref=send_buf.at[slot],
                dst_ref=recv_buf.at[slot],
                send_sem=send_sems.at[slot],
                recv_sem=recv_sems.at[slot],
                device_id=(partners[dim],),
                device_id_type=pl.DeviceIdType.MESH,
            )

        x_val = x_ref[...]
        for l, (wi, wo) in enumerate([(wi0, wo0), (wi1, wo1), (wi2, wo2)]):
            partial = jnp.dot(x_val, wi[...], preferred_element_type=jnp.float32)
            acc = [partial[q * bq:(q + 1) * bq, :] for q in range(N_Q)]
            for q in range(N_Q):
                slot, rdma = make_rdma(l, 0, q)
                send_buf[slot, :, :] = acc[q]
                rdma.start()
            for s in range(N_DIMS):
                for q in range(N_Q):
                    slot, rdma = make_rdma(l, s, q)
                    rdma.wait()
                    acc[q] = acc[q] + recv_buf[slot, :, :]
                    if s + 1 < N_DIMS:
                        nslot, nrdma = make_rdma(l, s + 1, q)
                        send_buf[nslot, :, :] = acc[q]
                        nrdma.start()
            h = jnp.maximum(jnp.concatenate(acc, axis=0), 0.0)
            x_val = jnp.dot(h, wo[...], preferred_element_type=jnp.float32)
        out_ref[...] = x_val


    return pl.pallas_call(
        body,
        out_shape=jax.ShapeDtypeStruct((b, d_shard), jnp.float32),
        in_specs=[pl.BlockSpec(memory_space=pltpu.VMEM)] * 7,
        out_specs=pl.BlockSpec(memory_space=pltpu.VMEM),
        scratch_shapes=[
            pltpu.VMEM((N_SLOTS, bq, h_dim), jnp.float32),
            pltpu.VMEM((N_SLOTS, bq, h_dim), jnp.float32),
            pltpu.SemaphoreType.DMA((N_SLOTS,)),
            pltpu.SemaphoreType.DMA((N_SLOTS,)),
        ],
        compiler_params=pltpu.CompilerParams(collective_id=0),
    )(x, Win0, Wout0, Win1, Wout1, Win2, Wout2)
